# baseline (speedup 1.0000x reference)
import functools

import jax
import jax.numpy as jnp
import numpy as np
from jax import lax
from jax.experimental import pallas as pl
from jax.experimental.pallas import tpu as pltpu
from jax.experimental.pallas import tpu_sc as plsc

PI = 3.141592653589793
X_MIN, X_N = 1e-05, 5000
SIGMA_MIN, SIGMA_MAX, SIGMA_N = 0.003, 2, 5000

_N = 16777216
_TC_GRID = 32
_TC_CHUNK = _N // _TC_GRID

_PAD_ROWS, _PAD_COLS = 5008, 5120
_TABLE_WORDS = _PAD_ROWS * _PAD_COLS

_FLIP_BIT = 1 << 25
_OFF_MASK = _FLIP_BIT - 1

_NC, _NS = 2, 16
_NW = _NC * _NS


def _bin_kernel(x_ref, s_ref, idx_ref):
    x = x_ref[...]
    sigma = s_ref[...]
    xw = (x + PI) % (2 * PI) - PI
    xl = jnp.log(jnp.abs(xw) / PI + 1e-10)
    xi = (xl + np.float32(-np.log(X_MIN))) * np.float32(X_N / -np.log(X_MIN))
    xi = jnp.round(jnp.clip(xi, 0, X_N)).astype(jnp.int32)
    sl = jnp.log(sigma / PI)
    si = ((sl + np.float32(-np.log(SIGMA_MIN)))
          * np.float32(SIGMA_N / (np.log(SIGMA_MAX) - np.log(SIGMA_MIN))))
    si = jnp.round(jnp.clip(si, 0, SIGMA_N)).astype(jnp.int32)
    xi = jnp.where(xw == 0.0, jnp.int32(X_N + 1), xi)
    off = (((si >> 3) * (_PAD_COLS // 128) + (xi >> 7)) * 1024
           + ((si & 7) << 7) + (xi & 127))
    flip = jnp.where(xw > 0.0, jnp.int32(_FLIP_BIT), jnp.int32(0))
    idx_ref[...] = off | flip


_S = 4
_NSLICE = _N // _S
_S_GRID = _TC_GRID // _S
_EPW = _NSLICE // _NW
_SC_CHUNK = 16384
_N_CHUNKS = _EPW // _SC_CHUNK


def _binning_slice(x, sigma, s):
    return pl.pallas_call(
        _bin_kernel,
        grid=(_S_GRID,),
        in_specs=[
            pl.BlockSpec((_TC_CHUNK,), lambda i: (s * _S_GRID + i,)),
            pl.BlockSpec((_TC_CHUNK,), lambda i: (s * _S_GRID + i,)),
        ],
        out_specs=pl.BlockSpec((_TC_CHUNK,), lambda i: (i,)),
        out_shape=jax.ShapeDtypeStruct((_NSLICE,), jnp.int32),
    )(x, sigma)


def _sc_gather_body(table_ref, idx_hbm, out_hbm,
                    idx0, idx1, mask0, mask1, vals0, vals1,
                    sin0, sin1, sg0, sg1, sout0, sout1):
    idxs, masks, vals = (idx0, idx1), (mask0, mask1), (vals0, vals1)
    sins, sgs, souts = (sin0, sin1), (sg0, sg1), (sout0, sout1)
    wid = lax.axis_index("s") * _NC + lax.axis_index("c")
    base = wid * _EPW

    def start_in(c, b):
        o = base + c * _SC_CHUNK
        pltpu.async_copy(idx_hbm.at[pl.ds(o, _SC_CHUNK)], idxs[b], sins[b])

    def wait_in(b):
        pltpu.make_async_copy(idx_hbm.at[pl.ds(base, _SC_CHUNK)], idxs[b],
                              sins[b]).wait()

    def strip(b):
        iv, mv = idxs[b], masks[b]

        def sbody(t, _):
            for u in range(8):
                s = pl.ds((t * 8 + u) * 16, 16)
                raw = iv[s]
                mv[s] = (raw & _FLIP_BIT) << 6
                iv[s] = raw & _OFF_MASK
            return ()

        lax.fori_loop(0, _SC_CHUNK // 128, sbody, ())

    def gather_start(b):
        pltpu.async_copy(table_ref.at[idxs[b]], vals[b], sgs[b])

    def gather_wait(b):
        pltpu.make_async_copy(table_ref.at[idxs[b]], vals[b], sgs[b]).wait()

    def out_start(c, b):
        o = base + c * _SC_CHUNK
        pltpu.async_copy(vals[b], out_hbm.at[pl.ds(o, _SC_CHUNK)], souts[b])

    def out_wait(b):
        pltpu.make_async_copy(vals[b], out_hbm.at[pl.ds(base, _SC_CHUNK)],
                              souts[b]).wait()

    def apply_sign(b):
        vv, mv = vals[b], masks[b]

        def mbody(t, _):
            for u in range(8):
                s = pl.ds((t * 8 + u) * 16, 16)
                bits = lax.bitcast_convert_type(vv[s], jnp.int32) ^ mv[s]
                vv[s] = lax.bitcast_convert_type(bits, jnp.float32)
            return ()

        lax.fori_loop(0, _SC_CHUNK // 128, mbody, ())

    start_in(0, 0)
    wait_in(0)
    strip(0)
    gather_start(0)
    start_in(1, 1)

    def loop_body(i, _):
        for b in (0, 1):
            c = 2 * i + b
            nb = 1 - b

            @pl.when(jnp.logical_and(c + 1 < _N_CHUNKS, c >= 1))
            def _():
                out_wait(nb)

            @pl.when(c + 1 < _N_CHUNKS)
            def _():
                wait_in(nb)
                strip(nb)
                gather_start(nb)

            gather_wait(b)

            @pl.when(c + 2 < _N_CHUNKS)
            def _():
                start_in(c + 2, b)

            apply_sign(b)
            out_start(c, b)
        return ()

    lax.fori_loop(0, _N_CHUNKS // 2, loop_body, ())
    out_wait(0)
    out_wait(1)


@functools.lru_cache(maxsize=None)
def _make_sc_gather():
    return pl.kernel(
        _sc_gather_body,
        out_type=jax.ShapeDtypeStruct((_NSLICE,), jnp.float32),
        mesh=plsc.VectorSubcoreMesh(core_axis_name="c", subcore_axis_name="s",
                                    num_cores=_NC, num_subcores=_NS),
        scratch_types=[
            pltpu.VMEM((_SC_CHUNK,), jnp.int32),
            pltpu.VMEM((_SC_CHUNK,), jnp.int32),
            pltpu.VMEM((_SC_CHUNK,), jnp.int32),
            pltpu.VMEM((_SC_CHUNK,), jnp.int32),
            pltpu.VMEM((_SC_CHUNK,), jnp.float32),
            pltpu.VMEM((_SC_CHUNK,), jnp.float32),
            pltpu.SemaphoreType.DMA,
            pltpu.SemaphoreType.DMA,
            pltpu.SemaphoreType.DMA,
            pltpu.SemaphoreType.DMA,
            pltpu.SemaphoreType.DMA,
            pltpu.SemaphoreType.DMA,
        ],
    )


def kernel(x, sigma, score_table):
    t4 = jnp.pad(score_table,
                 ((0, _PAD_ROWS - (SIGMA_N + 1)),
                  (0, _PAD_COLS - (X_N + 1))))
    table_flat = (t4.reshape(_PAD_ROWS // 8, 8, _PAD_COLS // 128, 128)
                  .transpose(0, 2, 1, 3).reshape(-1))
    sc = _make_sc_gather()
    outs = []
    for s in range(_S):
        idx_s = _binning_slice(x, sigma, s)
        outs.append(sc(table_flat, idx_s))
    return jnp.concatenate(outs)


# speedup vs baseline: 1.0551x; 1.0551x over previous
import functools

import jax
import jax.numpy as jnp
import numpy as np
from jax import lax
from jax.experimental import pallas as pl
from jax.experimental.pallas import tpu as pltpu
from jax.experimental.pallas import tpu_sc as plsc

PI = 3.141592653589793
X_MIN, X_N = 1e-05, 5000
SIGMA_MIN, SIGMA_MAX, SIGMA_N = 0.003, 2, 5000

_N = 16777216
_TC_GRID = 32
_TC_CHUNK = _N // _TC_GRID

_PAD_ROWS, _PAD_COLS = 5008, 5120
_TABLE_WORDS = _PAD_ROWS * _PAD_COLS

_FLIP_BIT = 1 << 25
_OFF_MASK = _FLIP_BIT - 1

_NC, _NS = 2, 16
_NW = _NC * _NS
_EPW = _N // _NW
_SC_CHUNK = 16384
_N_CHUNKS = _EPW // _SC_CHUNK


def _bin_kernel(x_ref, s_ref, idx_ref):
    x = x_ref[...]
    sigma = s_ref[...]
    xw = (x + PI) % (2 * PI) - PI
    xl = jnp.log(jnp.abs(xw) / PI + 1e-10)
    xi = (xl + np.float32(-np.log(X_MIN))) * np.float32(X_N / -np.log(X_MIN))
    xi = jnp.round(jnp.clip(xi, 0, X_N)).astype(jnp.int32)
    sl = jnp.log(sigma / PI)
    si = ((sl + np.float32(-np.log(SIGMA_MIN)))
          * np.float32(SIGMA_N / (np.log(SIGMA_MAX) - np.log(SIGMA_MIN))))
    si = jnp.round(jnp.clip(si, 0, SIGMA_N)).astype(jnp.int32)
    xi = jnp.where(xw == 0.0, jnp.int32(X_N + 1), xi)
    off = (((si >> 3) * (_PAD_COLS // 128) + (xi >> 7)) * 1024
           + ((si & 7) << 7) + (xi & 127))
    flip = jnp.where(xw > 0.0, jnp.int32(_FLIP_BIT), jnp.int32(0))
    idx_ref[...] = off | flip


def _binning(x, sigma):
    return pl.pallas_call(
        _bin_kernel,
        grid=(_TC_GRID,),
        in_specs=[
            pl.BlockSpec((_TC_CHUNK,), lambda i: (i,)),
            pl.BlockSpec((_TC_CHUNK,), lambda i: (i,)),
        ],
        out_specs=pl.BlockSpec((_TC_CHUNK,), lambda i: (i,)),
        out_shape=jax.ShapeDtypeStruct((_N,), jnp.int32),
    )(x, sigma)


def _sc_gather_body(table_ref, idx_hbm, out_hbm,
                    raw0, raw1, sidx0, sidx1, vals0, vals1,
                    sin0, sin1, sg0, sg1, sout0, sout1):
    raws, sidxs, vals = (raw0, raw1), (sidx0, sidx1), (vals0, vals1)
    sins, sgs, souts = (sin0, sin1), (sg0, sg1), (sout0, sout1)
    wid = lax.axis_index("s") * _NC + lax.axis_index("c")
    base = wid * _EPW

    def start_in(c, b):
        o = base + c * _SC_CHUNK
        pltpu.async_copy(idx_hbm.at[pl.ds(o, _SC_CHUNK)], raws[b], sins[b])

    def wait_in(b):
        pltpu.make_async_copy(idx_hbm.at[pl.ds(base, _SC_CHUNK)], raws[b],
                              sins[b]).wait()

    def strip(b):
        rv, sv = raws[b], sidxs[b]

        def sbody(t, _):
            for u in range(8):
                s = pl.ds((t * 8 + u) * 16, 16)
                sv[s] = rv[s] & _OFF_MASK
            return ()

        lax.fori_loop(0, _SC_CHUNK // 128, sbody, ())

    def gather_start(b):
        pltpu.async_copy(table_ref.at[sidxs[b]], vals[b], sgs[b])

    def gather_wait(b):
        pltpu.make_async_copy(table_ref.at[sidxs[b]], vals[b], sgs[b]).wait()

    def out_start(c, b):
        o = base + c * _SC_CHUNK
        pltpu.async_copy(vals[b], out_hbm.at[pl.ds(o, _SC_CHUNK)], souts[b])

    def out_wait(b):
        pltpu.make_async_copy(vals[b], out_hbm.at[pl.ds(base, _SC_CHUNK)],
                              souts[b]).wait()

    def apply_sign(b):
        vv, rv = vals[b], raws[b]

        def mbody(t, _):
            for u in range(8):
                s = pl.ds((t * 8 + u) * 16, 16)
                mask = (rv[s] & _FLIP_BIT) << 6
                bits = lax.bitcast_convert_type(vv[s], jnp.int32) ^ mask
                vv[s] = lax.bitcast_convert_type(bits, jnp.float32)
            return ()

        lax.fori_loop(0, _SC_CHUNK // 128, mbody, ())

    start_in(0, 0)
    wait_in(0)
    strip(0)
    gather_start(0)
    start_in(1, 1)

    def loop_body(i, _):
        for b in (0, 1):
            c = 2 * i + b
            nb = 1 - b

            @pl.when(jnp.logical_and(c + 1 < _N_CHUNKS, c >= 1))
            def _():
                out_wait(nb)

            @pl.when(c + 1 < _N_CHUNKS)
            def _():
                wait_in(nb)
                strip(nb)
                gather_start(nb)

            gather_wait(b)
            apply_sign(b)
            out_start(c, b)

            @pl.when(c + 2 < _N_CHUNKS)
            def _():
                start_in(c + 2, b)
        return ()

    lax.fori_loop(0, _N_CHUNKS // 2, loop_body, ())
    out_wait(0)
    out_wait(1)


@functools.lru_cache(maxsize=None)
def _make_sc_gather():
    return pl.kernel(
        _sc_gather_body,
        out_type=jax.ShapeDtypeStruct((_N,), jnp.float32),
        mesh=plsc.VectorSubcoreMesh(core_axis_name="c", subcore_axis_name="s",
                                    num_cores=_NC, num_subcores=_NS),
        scratch_types=[
            pltpu.VMEM((_SC_CHUNK,), jnp.int32),
            pltpu.VMEM((_SC_CHUNK,), jnp.int32),
            pltpu.VMEM((_SC_CHUNK,), jnp.int32),
            pltpu.VMEM((_SC_CHUNK,), jnp.int32),
            pltpu.VMEM((_SC_CHUNK,), jnp.float32),
            pltpu.VMEM((_SC_CHUNK,), jnp.float32),
            pltpu.SemaphoreType.DMA,
            pltpu.SemaphoreType.DMA,
            pltpu.SemaphoreType.DMA,
            pltpu.SemaphoreType.DMA,
            pltpu.SemaphoreType.DMA,
            pltpu.SemaphoreType.DMA,
        ],
    )


def kernel(x, sigma, score_table):
    idx = _binning(x, sigma)
    t4 = jnp.pad(score_table,
                 ((0, _PAD_ROWS - (SIGMA_N + 1)),
                  (0, _PAD_COLS - (X_N + 1))))
    table_flat = (t4.reshape(_PAD_ROWS // 8, 8, _PAD_COLS // 128, 128)
                  .transpose(0, 2, 1, 3).reshape(-1))
    return _make_sc_gather()(table_flat, idx)


# speedup vs baseline: 1.0560x; 1.0008x over previous
import functools

import jax
import jax.numpy as jnp
import numpy as np
from jax import lax
from jax.experimental import pallas as pl
from jax.experimental.pallas import tpu as pltpu
from jax.experimental.pallas import tpu_sc as plsc

PI = 3.141592653589793
X_MIN, X_N = 1e-05, 5000
SIGMA_MIN, SIGMA_MAX, SIGMA_N = 0.003, 2, 5000

_N = 16777216
_TC_GRID = 32
_TC_CHUNK = _N // _TC_GRID

_PAD_ROWS, _PAD_COLS = 5008, 5120
_TABLE_WORDS = _PAD_ROWS * _PAD_COLS

_FLIP_BIT = 1 << 25
_OFF_MASK = _FLIP_BIT - 1

_NC, _NS = 2, 16
_NW = _NC * _NS
_EPW = _N // _NW
_SC_CHUNK = 16384
_N_CHUNKS = _EPW // _SC_CHUNK


def _bin_kernel(x_ref, s_ref, idx_ref):
    x = x_ref[...]
    sigma = s_ref[...]
    xw = (x + PI) % (2 * PI) - PI
    xl = jnp.log(jnp.abs(xw) / PI + 1e-10)
    xi = (xl + np.float32(-np.log(X_MIN))) * np.float32(X_N / -np.log(X_MIN))
    xi = jnp.round(jnp.clip(xi, 0, X_N)).astype(jnp.int32)
    sl = jnp.log(sigma / PI)
    si = ((sl + np.float32(-np.log(SIGMA_MIN)))
          * np.float32(SIGMA_N / (np.log(SIGMA_MAX) - np.log(SIGMA_MIN))))
    si = jnp.round(jnp.clip(si, 0, SIGMA_N)).astype(jnp.int32)
    xi = jnp.where(xw == 0.0, jnp.int32(X_N + 1), xi)
    off = (((si >> 3) * (_PAD_COLS // 128) + (xi >> 7)) * 1024
           + ((si & 7) << 7) + (xi & 127))
    flip = jnp.where(xw > 0.0, jnp.int32(_FLIP_BIT), jnp.int32(0))
    idx_ref[...] = off | flip


def _binning(x, sigma):
    return pl.pallas_call(
        _bin_kernel,
        grid=(_TC_GRID,),
        in_specs=[
            pl.BlockSpec((_TC_CHUNK,), lambda i: (i,)),
            pl.BlockSpec((_TC_CHUNK,), lambda i: (i,)),
        ],
        out_specs=pl.BlockSpec((_TC_CHUNK,), lambda i: (i,)),
        out_shape=jax.ShapeDtypeStruct((_N,), jnp.int32),
    )(x, sigma)


def _sc_gather_body(table_ref, idx_hbm, out_hbm,
                    raw0, raw1, sidx0, sidx1, vals0, vals1,
                    sin0, sin1, sg0, sg1, sout0, sout1):
    raws, sidxs, vals = (raw0, raw1), (sidx0, sidx1), (vals0, vals1)
    sins, sgs, souts = (sin0, sin1), (sg0, sg1), (sout0, sout1)
    wid = lax.axis_index("s") * _NC + lax.axis_index("c")
    base = wid * _EPW

    def start_in(c, b):
        o = base + c * _SC_CHUNK
        pltpu.async_copy(idx_hbm.at[pl.ds(o, _SC_CHUNK)], raws[b], sins[b])

    def wait_in(b):
        pltpu.make_async_copy(idx_hbm.at[pl.ds(base, _SC_CHUNK)], raws[b],
                              sins[b]).wait()

    def strip(b):
        rv, sv = raws[b], sidxs[b]

        def sbody(t, _):
            for u in range(8):
                s = pl.ds((t * 8 + u) * 16, 16)
                sv[s] = rv[s] & _OFF_MASK
            return ()

        lax.fori_loop(0, _SC_CHUNK // 128, sbody, ())

    _H = _SC_CHUNK // 2

    def gather_start(b):
        pltpu.async_copy(table_ref.at[sidxs[b].at[pl.ds(0, _H)]],
                         vals[b].at[pl.ds(0, _H)], sgs[b])
        pltpu.async_copy(table_ref.at[sidxs[b].at[pl.ds(_H, _H)]],
                         vals[b].at[pl.ds(_H, _H)], sgs[b])

    def gather_wait(b):
        pltpu.make_async_copy(table_ref.at[sidxs[b].at[pl.ds(0, _H)]],
                              vals[b].at[pl.ds(0, _H)], sgs[b]).wait()
        pltpu.make_async_copy(table_ref.at[sidxs[b].at[pl.ds(_H, _H)]],
                              vals[b].at[pl.ds(_H, _H)], sgs[b]).wait()

    def out_start(c, b):
        o = base + c * _SC_CHUNK
        pltpu.async_copy(vals[b], out_hbm.at[pl.ds(o, _SC_CHUNK)], souts[b])

    def out_wait(b):
        pltpu.make_async_copy(vals[b], out_hbm.at[pl.ds(base, _SC_CHUNK)],
                              souts[b]).wait()

    def apply_sign(b):
        vv, rv = vals[b], raws[b]

        def mbody(t, _):
            for u in range(8):
                s = pl.ds((t * 8 + u) * 16, 16)
                mask = (rv[s] & _FLIP_BIT) << 6
                bits = lax.bitcast_convert_type(vv[s], jnp.int32) ^ mask
                vv[s] = lax.bitcast_convert_type(bits, jnp.float32)
            return ()

        lax.fori_loop(0, _SC_CHUNK // 128, mbody, ())

    start_in(0, 0)
    wait_in(0)
    strip(0)
    gather_start(0)
    start_in(1, 1)

    def loop_body(i, _):
        for b in (0, 1):
            c = 2 * i + b
            nb = 1 - b

            @pl.when(jnp.logical_and(c + 1 < _N_CHUNKS, c >= 1))
            def _():
                out_wait(nb)

            @pl.when(c + 1 < _N_CHUNKS)
            def _():
                wait_in(nb)
                strip(nb)
                gather_start(nb)

            gather_wait(b)
            apply_sign(b)
            out_start(c, b)

            @pl.when(c + 2 < _N_CHUNKS)
            def _():
                start_in(c + 2, b)
        return ()

    lax.fori_loop(0, _N_CHUNKS // 2, loop_body, ())
    out_wait(0)
    out_wait(1)


@functools.lru_cache(maxsize=None)
def _make_sc_gather():
    return pl.kernel(
        _sc_gather_body,
        out_type=jax.ShapeDtypeStruct((_N,), jnp.float32),
        mesh=plsc.VectorSubcoreMesh(core_axis_name="c", subcore_axis_name="s",
                                    num_cores=_NC, num_subcores=_NS),
        scratch_types=[
            pltpu.VMEM((_SC_CHUNK,), jnp.int32),
            pltpu.VMEM((_SC_CHUNK,), jnp.int32),
            pltpu.VMEM((_SC_CHUNK,), jnp.int32),
            pltpu.VMEM((_SC_CHUNK,), jnp.int32),
            pltpu.VMEM((_SC_CHUNK,), jnp.float32),
            pltpu.VMEM((_SC_CHUNK,), jnp.float32),
            pltpu.SemaphoreType.DMA,
            pltpu.SemaphoreType.DMA,
            pltpu.SemaphoreType.DMA,
            pltpu.SemaphoreType.DMA,
            pltpu.SemaphoreType.DMA,
            pltpu.SemaphoreType.DMA,
        ],
    )


def kernel(x, sigma, score_table):
    idx = _binning(x, sigma)
    t4 = jnp.pad(score_table,
                 ((0, _PAD_ROWS - (SIGMA_N + 1)),
                  (0, _PAD_COLS - (X_N + 1))))
    table_flat = (t4.reshape(_PAD_ROWS // 8, 8, _PAD_COLS // 128, 128)
                  .transpose(0, 2, 1, 3).reshape(-1))
    return _make_sc_gather()(table_flat, idx)
